# baseline (speedup 1.0000x reference)
import functools

import jax
import jax.numpy as jnp
from jax.experimental import pallas as pl
from jax.experimental.pallas import tpu as pltpu

EPS = 1e-5
_VMEM_LIMIT = 100 * 1024 * 1024


def _conv1_kernel(xt_ref, w1_ref, wd_ref, z_ref, st_ref, xpad_ref, pl_ref,
                  *, Nb, Ho, Wo, Cin):
    H, W = 2 * Ho, 2 * Wo
    Ap = Ho + 1
    Wp, Hp = Wo + 2, Ho + 2
    P = Hp * Wp
    Rp = Nb * P
    L1 = Rp - 2 * Wp

    xpad_ref[...] = jnp.zeros_like(xpad_ref)
    xpad_ref[:, 1:1 + H, 1:1 + W, :] = xt_ref[...]
    x5 = xpad_ref[...].reshape(Nb, Ap, 2, Ap, 2, Cin)

    pl_ref[...] = jnp.zeros_like(pl_ref)
    for pi in (0, 1):
        for pj in (0, 1):
            pl_ref[pi * 2 + pj, :, 0:Ap, 0:Ap, :] = x5[:, :, pi, :, pj, :]
    planes = [pl_ref[p].reshape(Rp, Cin) for p in range(4)]

    z1 = None
    for dy in range(3):
        for dx in range(3):
            p = (dy % 2) * 2 + (dx % 2)
            base = (dy // 2) * Wp + (dx // 2)
            t = planes[p][base:base + L1, :]
            d = jnp.dot(t, w1_ref[dy * 3 + dx].astype(jnp.bfloat16),
                        preferred_element_type=jnp.float32)
            z1 = d if z1 is None else z1 + d
    tc = planes[3][0:L1, :]
    zd = jnp.dot(tc, wd_ref[...].astype(jnp.bfloat16),
                 preferred_element_type=jnp.float32)
    z = jnp.concatenate([z1, zd], axis=-1)

    rows = jax.lax.broadcasted_iota(jnp.int32, (L1, 1), 0)
    valid = ((rows % Wp) < Wo) & ((rows % P) < Ho * Wp)
    zv = jnp.where(valid, z, 0.0)
    zfull = jnp.concatenate(
        [zv, jnp.zeros((Rp - L1, zv.shape[-1]), jnp.float32)], axis=0)
    z_ref[...] = zfull.reshape(z_ref.shape).astype(z_ref.dtype)
    st_ref[0, 0:1, :] = jnp.sum(zv, axis=0, keepdims=True)
    st_ref[0, 1:2, :] = jnp.sum(zv * zv, axis=0, keepdims=True)
    st_ref[0, 2:8, :] = jnp.zeros_like(st_ref[0, 2:8, :])


def _conv2_kernel(z_ref, w2_ref, g1_ref, b1_ref, s1_ref, z2_ref, st_ref, yflat_ref,
                  *, Nb, Ho, Wo, C, inv_m):
    Wp, Hp = Wo + 2, Ho + 2
    P = Hp * Wp
    Rp = Nb * P
    ofs0 = Wp + 1

    s1 = jnp.sum(s1_ref[...], axis=0)
    mean = s1[0:1, :] * inv_m
    var = jnp.maximum(s1[1:2, :] * inv_m - mean * mean, 0.0)
    scale = jax.lax.rsqrt(var + EPS) * g1_ref[...]
    shift = b1_ref[...] - mean * scale

    rows = jax.lax.broadcasted_iota(jnp.int32, (Rp, 1), 0)
    valid = ((rows % Wp) < Wo) & ((rows % P) < Ho * Wp)
    zflat = z_ref[...].reshape(Rp, -1)
    y1 = jnp.maximum(zflat.astype(jnp.float32) * scale + shift, 0.0)
    yflat_ref[0:ofs0, :] = jnp.zeros_like(yflat_ref[0:ofs0, :])
    yflat_ref[ofs0:ofs0 + Rp, :] = (
        jnp.where(valid, y1, 0.0).astype(yflat_ref.dtype))
    yflat_ref[ofs0 + Rp:, :] = jnp.zeros_like(yflat_ref[ofs0 + Rp:, :])

    z2 = None
    for dy in range(3):
        for dx in range(3):
            t = yflat_ref[dy * Wp + dx:dy * Wp + dx + Rp, :]
            d = jnp.dot(t, w2_ref[dy * 3 + dx].astype(jnp.bfloat16),
                        preferred_element_type=jnp.float32)
            z2 = d if z2 is None else z2 + d
    zv = jnp.where(valid, z2, 0.0)
    z2_ref[...] = zv.reshape(z2_ref.shape).astype(z2_ref.dtype)
    st_ref[0, 0:1, :] = jnp.sum(zv, axis=0, keepdims=True)
    st_ref[0, 1:2, :] = jnp.sum(zv * zv, axis=0, keepdims=True)
    st_ref[0, 2:8, :] = jnp.zeros_like(st_ref[0, 2:8, :])


def _final_kernel(z2_ref, zd_ref, s1_ref, s2_ref, g2_ref, b2_ref,
                  gd_ref, bd_ref, o_ref, *,
                  Nb, Ho, Wo, inv_m):
    Wp, Hp = Wo + 2, Ho + 2
    P = Hp * Wp
    Rp = Nb * P
    Lq = Rp - 2 * Wp

    s1 = jnp.sum(s1_ref[...], axis=0)
    s2 = jnp.sum(s2_ref[...], axis=0)

    z2f = z2_ref[...].reshape(Rp, -1)
    zdf = zd_ref[...].reshape(Rp, -1)

    m2 = s2[0:1, :] * inv_m
    v2 = jnp.maximum(s2[1:2, :] * inv_m - m2 * m2, 0.0)
    sc2 = jax.lax.rsqrt(v2 + EPS) * g2_ref[...]
    y2 = z2f[0:Lq, :].astype(jnp.float32) * sc2 + (b2_ref[...] - m2 * sc2)

    md = s1[0:1, :] * inv_m
    vd = jnp.maximum(s1[1:2, :] * inv_m - md * md, 0.0)
    scd = jax.lax.rsqrt(vd + EPS) * gd_ref[...]
    yd = (zdf[0:Lq, :].astype(jnp.float32) * scd
          + (bd_ref[...] - md * scd))

    o = jnp.maximum(y2 + yd, 0.0)
    for k in range(Nb):
        v = o[k * P:k * P + Ho * Wp, :].reshape(Ho, Wp, -1)[:, 0:Wo, :]
        o_ref[k * Ho * Wo:(k + 1) * Ho * Wo, :] = v.reshape(Ho * Wo, -1)


def _block_impl(x, w1, g1, b1, w2, g2, b2, wd, gd, bd):
    N, Cin, H, W = x.shape
    C = w1.shape[-1]
    assert H % 2 == 0 and W % 2 == 0
    assert Cin % 128 == 0 and C % 128 == 0
    Ho, Wo = H // 2, W // 2
    Ap = Ho + 1
    M = N * Ho * Wo
    inv_m = 1.0 / M
    Nb = next(t for t in (4, 2, 1) if N % t == 0)
    nb = N // Nb
    R = Nb * Ho * Wo
    Hp, Wp = Ho + 2, Wo + 2
    P = Hp * Wp
    Rp = Nb * P

    xt = jnp.transpose(x.astype(jnp.bfloat16), (0, 2, 3, 1))

    w1r = w1.reshape(9, Cin, C)
    wdr = wd.reshape(Cin, C)
    w2r = w2.reshape(9, C, C)
    g1r, b1r = g1.reshape(1, C), b1.reshape(1, C)
    g2r, b2r = g2.reshape(1, C), b2.reshape(1, C)
    gdr, bdr = gd.reshape(1, C), bd.reshape(1, C)

    par = pltpu.CompilerParams(dimension_semantics=("parallel",),
                               vmem_limit_bytes=_VMEM_LIMIT)

    z, s1b = pl.pallas_call(
        functools.partial(_conv1_kernel, Nb=Nb, Ho=Ho, Wo=Wo, Cin=Cin),
        grid=(nb,),
        in_specs=[
            pl.BlockSpec((Nb, H, W, Cin), lambda i: (i, 0, 0, 0)),
            pl.BlockSpec((9, Cin, C), lambda i: (0, 0, 0)),
            pl.BlockSpec((Cin, C), lambda i: (0, 0)),
        ],
        out_specs=[
            pl.BlockSpec((Nb, Hp, Wp, 2 * C), lambda i: (i, 0, 0, 0)),
            pl.BlockSpec((1, 8, 2 * C), lambda i: (i, 0, 0)),
        ],
        out_shape=[
            jax.ShapeDtypeStruct((N, Hp, Wp, 2 * C), jnp.bfloat16),
            jax.ShapeDtypeStruct((nb, 8, 2 * C), jnp.float32),
        ],
        scratch_shapes=[
            pltpu.VMEM((Nb, 2 * Ap, 2 * Ap, Cin), jnp.bfloat16),
            pltpu.VMEM((4, Nb, Hp, Wp, Cin), jnp.bfloat16),
        ],
        compiler_params=par,
    )(xt, w1r, wdr)

    z2, s2b = pl.pallas_call(
        functools.partial(_conv2_kernel, Nb=Nb, Ho=Ho, Wo=Wo, C=C, inv_m=inv_m),
        grid=(nb,),
        in_specs=[
            pl.BlockSpec((Nb, Hp, Wp, C), lambda i: (i, 0, 0, 0)),
            pl.BlockSpec((9, C, C), lambda i: (0, 0, 0)),
            pl.BlockSpec((1, C), lambda i: (0, 0)),
            pl.BlockSpec((1, C), lambda i: (0, 0)),
            pl.BlockSpec((nb, 8, C), lambda i: (0, 0, 0)),
        ],
        out_specs=[
            pl.BlockSpec((Nb, Hp, Wp, C), lambda i: (i, 0, 0, 0)),
            pl.BlockSpec((1, 8, C), lambda i: (i, 0, 0)),
        ],
        out_shape=[
            jax.ShapeDtypeStruct((N, Hp, Wp, C), jnp.bfloat16),
            jax.ShapeDtypeStruct((nb, 8, C), jnp.float32),
        ],
        scratch_shapes=[
            pltpu.VMEM((Rp + 3 * Wp + 8, C), jnp.bfloat16)],
        compiler_params=par,
    )(z, w2r, g1r, b1r, s1b)

    out_flat = pl.pallas_call(
        functools.partial(_final_kernel, Nb=Nb, Ho=Ho, Wo=Wo, inv_m=inv_m),
        grid=(nb,),
        in_specs=[
            pl.BlockSpec((Nb, Hp, Wp, C), lambda i: (i, 0, 0, 0)),
            pl.BlockSpec((Nb, Hp, Wp, C), lambda i: (i, 0, 0, 1)),
            pl.BlockSpec((nb, 8, C), lambda i: (0, 0, 1)),
            pl.BlockSpec((nb, 8, C), lambda i: (0, 0, 0)),
            pl.BlockSpec((1, C), lambda i: (0, 0)),
            pl.BlockSpec((1, C), lambda i: (0, 0)),
            pl.BlockSpec((1, C), lambda i: (0, 0)),
            pl.BlockSpec((1, C), lambda i: (0, 0)),
        ],
        out_specs=pl.BlockSpec((R, C), lambda i: (i, 0)),
        out_shape=jax.ShapeDtypeStruct((M, C), jnp.float32),
        compiler_params=par,
    )(z2, z, s1b, s2b, g2r, b2r, gdr, bdr)

    return out_flat.reshape(N, Ho, Wo, C).transpose(0, 3, 1, 2)


def kernel(x, w1, g1, b1, w2, g2, b2, wd, gd, bd):
    return _block_impl(x, w1, g1, b1, w2, g2, b2, wd, gd, bd)


# speedup vs baseline: 1.0297x; 1.0297x over previous
import functools

import jax
import jax.numpy as jnp
from jax.experimental import pallas as pl
from jax.experimental.pallas import tpu as pltpu

EPS = 1e-5
_VMEM_LIMIT = 100 * 1024 * 1024


def _conv1_kernel(xt_ref, w1_ref, wd_ref, z_ref, st_ref, xpad_ref, pl_ref,
                  *, Nb, Ho, Wo, Cin):
    H, W = 2 * Ho, 2 * Wo
    Ap = Ho + 1
    Wp, Hp = Wo + 2, Ho + 2
    P = Hp * Wp
    Rp = Nb * P
    L1 = Rp - 2 * Wp

    xpad_ref[...] = jnp.zeros_like(xpad_ref)
    xpad_ref[:, 1:1 + H, 1:1 + W, :] = xt_ref[...]
    x5 = xpad_ref[...].reshape(Nb, Ap, 2, Ap, 2, Cin)

    pl_ref[...] = jnp.zeros_like(pl_ref)
    for pi in (0, 1):
        for pj in (0, 1):
            pl_ref[pi * 2 + pj, :, 0:Ap, 0:Ap, :] = x5[:, :, pi, :, pj, :]
    planes = [pl_ref[p].reshape(Rp, Cin) for p in range(4)]

    z1 = None
    for dy in range(3):
        for dx in range(3):
            p = (dy % 2) * 2 + (dx % 2)
            base = (dy // 2) * Wp + (dx // 2)
            t = planes[p][base:base + L1, :]
            d = jnp.dot(t, w1_ref[dy * 3 + dx].astype(jnp.bfloat16),
                        preferred_element_type=jnp.float32)
            z1 = d if z1 is None else z1 + d
    tc = planes[3][0:L1, :]
    zd = jnp.dot(tc, wd_ref[...].astype(jnp.bfloat16),
                 preferred_element_type=jnp.float32)
    z = jnp.concatenate([z1, zd], axis=-1)

    rows = jax.lax.broadcasted_iota(jnp.int32, (L1, 1), 0)
    valid = ((rows % Wp) < Wo) & ((rows % P) < Ho * Wp)
    zv = jnp.where(valid, z, 0.0)
    zfull = jnp.concatenate(
        [zv, jnp.zeros((Rp - L1, zv.shape[-1]), jnp.float32)], axis=0)
    z_ref[...] = zfull.reshape(z_ref.shape).astype(z_ref.dtype)
    st_ref[0, 0:1, :] = jnp.sum(zv, axis=0, keepdims=True)
    st_ref[0, 1:2, :] = jnp.sum(zv * zv, axis=0, keepdims=True)
    st_ref[0, 2:8, :] = jnp.zeros_like(st_ref[0, 2:8, :])


def _conv2_kernel(z_ref, w2_ref, g1_ref, b1_ref, s1_ref, z2_ref, st_ref, yflat_ref,
                  *, Nb, Ho, Wo, C, inv_m):
    Wp, Hp = Wo + 2, Ho + 2
    P = Hp * Wp
    Rp = Nb * P
    ofs0 = Wp + 1

    s1 = jnp.sum(s1_ref[...], axis=0)
    mean = s1[0:1, :] * inv_m
    var = jnp.maximum(s1[1:2, :] * inv_m - mean * mean, 0.0)
    scale = jax.lax.rsqrt(var + EPS) * g1_ref[...]
    shift = b1_ref[...] - mean * scale

    rows = jax.lax.broadcasted_iota(jnp.int32, (Rp, 1), 0)
    valid = ((rows % Wp) < Wo) & ((rows % P) < Ho * Wp)
    zflat = z_ref[...].reshape(Rp, -1)
    y1 = jnp.maximum(zflat.astype(jnp.float32) * scale + shift, 0.0)
    yflat_ref[0:ofs0, :] = jnp.zeros_like(yflat_ref[0:ofs0, :])
    yflat_ref[ofs0:ofs0 + Rp, :] = (
        jnp.where(valid, y1, 0.0).astype(yflat_ref.dtype))
    yflat_ref[ofs0 + Rp:, :] = jnp.zeros_like(yflat_ref[ofs0 + Rp:, :])

    z2 = None
    for dy in range(3):
        for dx in range(3):
            t = yflat_ref[dy * Wp + dx:dy * Wp + dx + Rp, :]
            d = jnp.dot(t, w2_ref[dy * 3 + dx].astype(jnp.bfloat16),
                        preferred_element_type=jnp.float32)
            z2 = d if z2 is None else z2 + d
    zv = jnp.where(valid, z2, 0.0)
    z2_ref[...] = zv.reshape(z2_ref.shape).astype(z2_ref.dtype)
    st_ref[0, 0:1, :] = jnp.sum(zv, axis=0, keepdims=True)
    st_ref[0, 1:2, :] = jnp.sum(zv * zv, axis=0, keepdims=True)
    st_ref[0, 2:8, :] = jnp.zeros_like(st_ref[0, 2:8, :])


def _final_kernel(z2_ref, zd_ref, s1_ref, s2_ref, g2_ref, b2_ref,
                  gd_ref, bd_ref, o_ref, *,
                  Nb, Ho, Wo, inv_m):
    Wp, Hp = Wo + 2, Ho + 2
    P = Hp * Wp
    Rp = Nb * P
    Lq = Rp - 2 * Wp

    s1 = jnp.sum(s1_ref[...], axis=0)
    s2 = jnp.sum(s2_ref[...], axis=0)

    z2f = z2_ref[...].reshape(Rp, -1)
    zdf = zd_ref[...].reshape(Rp, -1)

    m2 = s2[0:1, :] * inv_m
    v2 = jnp.maximum(s2[1:2, :] * inv_m - m2 * m2, 0.0)
    sc2 = jax.lax.rsqrt(v2 + EPS) * g2_ref[...]
    y2 = z2f[0:Lq, :].astype(jnp.float32) * sc2 + (b2_ref[...] - m2 * sc2)

    md = s1[0:1, :] * inv_m
    vd = jnp.maximum(s1[1:2, :] * inv_m - md * md, 0.0)
    scd = jax.lax.rsqrt(vd + EPS) * gd_ref[...]
    yd = (zdf[0:Lq, :].astype(jnp.float32) * scd
          + (bd_ref[...] - md * scd))

    o = jnp.maximum(y2 + yd, 0.0)
    for k in range(Nb):
        v = o[k * P:k * P + Ho * Wp, :].reshape(Ho, Wp, -1)[:, 0:Wo, :]
        o_ref[k * Ho * Wo:(k + 1) * Ho * Wo, :] = v.reshape(Ho * Wo, -1)


def _block_impl(x, w1, g1, b1, w2, g2, b2, wd, gd, bd):
    N, Cin, H, W = x.shape
    C = w1.shape[-1]
    assert H % 2 == 0 and W % 2 == 0
    assert Cin % 128 == 0 and C % 128 == 0
    Ho, Wo = H // 2, W // 2
    Ap = Ho + 1
    M = N * Ho * Wo
    inv_m = 1.0 / M
    Nb = next(t for t in (16, 8, 4, 2, 1) if N % t == 0)
    nb = N // Nb
    R = Nb * Ho * Wo
    Hp, Wp = Ho + 2, Wo + 2
    P = Hp * Wp
    Rp = Nb * P

    xt = jnp.transpose(x.astype(jnp.bfloat16), (0, 2, 3, 1))

    w1r = w1.reshape(9, Cin, C)
    wdr = wd.reshape(Cin, C)
    w2r = w2.reshape(9, C, C)
    g1r, b1r = g1.reshape(1, C), b1.reshape(1, C)
    g2r, b2r = g2.reshape(1, C), b2.reshape(1, C)
    gdr, bdr = gd.reshape(1, C), bd.reshape(1, C)

    par = pltpu.CompilerParams(dimension_semantics=("parallel",),
                               vmem_limit_bytes=_VMEM_LIMIT)

    z, s1b = pl.pallas_call(
        functools.partial(_conv1_kernel, Nb=Nb, Ho=Ho, Wo=Wo, Cin=Cin),
        grid=(nb,),
        in_specs=[
            pl.BlockSpec((Nb, H, W, Cin), lambda i: (i, 0, 0, 0)),
            pl.BlockSpec((9, Cin, C), lambda i: (0, 0, 0)),
            pl.BlockSpec((Cin, C), lambda i: (0, 0)),
        ],
        out_specs=[
            pl.BlockSpec((Nb, Hp, Wp, 2 * C), lambda i: (i, 0, 0, 0)),
            pl.BlockSpec((1, 8, 2 * C), lambda i: (i, 0, 0)),
        ],
        out_shape=[
            jax.ShapeDtypeStruct((N, Hp, Wp, 2 * C), jnp.bfloat16),
            jax.ShapeDtypeStruct((nb, 8, 2 * C), jnp.float32),
        ],
        scratch_shapes=[
            pltpu.VMEM((Nb, 2 * Ap, 2 * Ap, Cin), jnp.bfloat16),
            pltpu.VMEM((4, Nb, Hp, Wp, Cin), jnp.bfloat16),
        ],
        compiler_params=par,
    )(xt, w1r, wdr)

    z2, s2b = pl.pallas_call(
        functools.partial(_conv2_kernel, Nb=Nb, Ho=Ho, Wo=Wo, C=C, inv_m=inv_m),
        grid=(nb,),
        in_specs=[
            pl.BlockSpec((Nb, Hp, Wp, C), lambda i: (i, 0, 0, 0)),
            pl.BlockSpec((9, C, C), lambda i: (0, 0, 0)),
            pl.BlockSpec((1, C), lambda i: (0, 0)),
            pl.BlockSpec((1, C), lambda i: (0, 0)),
            pl.BlockSpec((nb, 8, C), lambda i: (0, 0, 0)),
        ],
        out_specs=[
            pl.BlockSpec((Nb, Hp, Wp, C), lambda i: (i, 0, 0, 0)),
            pl.BlockSpec((1, 8, C), lambda i: (i, 0, 0)),
        ],
        out_shape=[
            jax.ShapeDtypeStruct((N, Hp, Wp, C), jnp.bfloat16),
            jax.ShapeDtypeStruct((nb, 8, C), jnp.float32),
        ],
        scratch_shapes=[
            pltpu.VMEM((Rp + 3 * Wp + 8, C), jnp.bfloat16)],
        compiler_params=par,
    )(z, w2r, g1r, b1r, s1b)

    out_flat = pl.pallas_call(
        functools.partial(_final_kernel, Nb=Nb, Ho=Ho, Wo=Wo, inv_m=inv_m),
        grid=(nb,),
        in_specs=[
            pl.BlockSpec((Nb, Hp, Wp, C), lambda i: (i, 0, 0, 0)),
            pl.BlockSpec((Nb, Hp, Wp, C), lambda i: (i, 0, 0, 1)),
            pl.BlockSpec((nb, 8, C), lambda i: (0, 0, 1)),
            pl.BlockSpec((nb, 8, C), lambda i: (0, 0, 0)),
            pl.BlockSpec((1, C), lambda i: (0, 0)),
            pl.BlockSpec((1, C), lambda i: (0, 0)),
            pl.BlockSpec((1, C), lambda i: (0, 0)),
            pl.BlockSpec((1, C), lambda i: (0, 0)),
        ],
        out_specs=pl.BlockSpec((R, C), lambda i: (i, 0)),
        out_shape=jax.ShapeDtypeStruct((M, C), jnp.float32),
        compiler_params=par,
    )(z2, z, s1b, s2b, g2r, b2r, gdr, bdr)

    return out_flat.reshape(N, Ho, Wo, C).transpose(0, 3, 1, 2)


def kernel(x, w1, g1, b1, w2, g2, b2, wd, gd, bd):
    return _block_impl(x, w1, g1, b1, w2, g2, b2, wd, gd, bd)


# speedup vs baseline: 1.0327x; 1.0028x over previous
import functools

import jax
import jax.numpy as jnp
from jax.experimental import pallas as pl
from jax.experimental.pallas import tpu as pltpu

EPS = 1e-5
_VMEM_LIMIT = 100 * 1024 * 1024


def _conv1_kernel(xt_ref, w1_ref, wd_ref, z_ref, st_ref, xpad_ref, pl_ref,
                  *, Nb, Ho, Wo, Cin):
    H, W = 2 * Ho, 2 * Wo
    Ap = Ho + 1
    Wp, Hp = Wo + 2, Ho + 2
    P = Hp * Wp
    Rp = Nb * P
    L1 = Rp - 2 * Wp

    xpad_ref[...] = jnp.zeros_like(xpad_ref)
    xpad_ref[:, 1:1 + H, 1:1 + W, :] = xt_ref[...]
    x5 = xpad_ref[...].reshape(Nb, Ap, 2, Ap, 2, Cin)

    pl_ref[...] = jnp.zeros_like(pl_ref)
    for pi in (0, 1):
        for pj in (0, 1):
            pl_ref[pi * 2 + pj, :, 0:Ap, 0:Ap, :] = x5[:, :, pi, :, pj, :]
    planes = [pl_ref[p].reshape(Rp, Cin) for p in range(4)]

    z1 = None
    for dy in range(3):
        for dx in range(3):
            p = (dy % 2) * 2 + (dx % 2)
            base = (dy // 2) * Wp + (dx // 2)
            t = planes[p][base:base + L1, :]
            d = jnp.dot(t, w1_ref[dy * 3 + dx].astype(jnp.bfloat16),
                        preferred_element_type=jnp.float32)
            z1 = d if z1 is None else z1 + d
    tc = planes[3][0:L1, :]
    zd = jnp.dot(tc, wd_ref[...].astype(jnp.bfloat16),
                 preferred_element_type=jnp.float32)
    z = jnp.concatenate([z1, zd], axis=-1)

    rows = jax.lax.broadcasted_iota(jnp.int32, (L1, 1), 0)
    valid = ((rows % Wp) < Wo) & ((rows % P) < Ho * Wp)
    zv = jnp.where(valid, z, 0.0)
    zfull = jnp.concatenate(
        [zv, jnp.zeros((Rp - L1, zv.shape[-1]), jnp.float32)], axis=0)
    z_ref[...] = zfull.reshape(z_ref.shape).astype(z_ref.dtype)
    st_ref[0, 0:1, :] = jnp.sum(zv, axis=0, keepdims=True)
    st_ref[0, 1:2, :] = jnp.sum(zv * zv, axis=0, keepdims=True)
    st_ref[0, 2:8, :] = jnp.zeros_like(st_ref[0, 2:8, :])


def _conv2_kernel(z_ref, w2_ref, g1_ref, b1_ref, s1_ref, z2_ref, st_ref, yflat_ref,
                  *, Nb, Ho, Wo, C, inv_m):
    Wp, Hp = Wo + 2, Ho + 2
    P = Hp * Wp
    Rp = Nb * P
    ofs0 = Wp + 1

    s1 = jnp.sum(s1_ref[...], axis=0)
    mean = s1[0:1, :] * inv_m
    var = jnp.maximum(s1[1:2, :] * inv_m - mean * mean, 0.0)
    scale = jax.lax.rsqrt(var + EPS) * g1_ref[...]
    shift = b1_ref[...] - mean * scale

    rows = jax.lax.broadcasted_iota(jnp.int32, (Rp, 1), 0)
    valid = ((rows % Wp) < Wo) & ((rows % P) < Ho * Wp)
    zflat = z_ref[...].reshape(Rp, -1)
    y1 = jnp.maximum(zflat.astype(jnp.float32) * scale + shift, 0.0)
    yflat_ref[0:ofs0, :] = jnp.zeros_like(yflat_ref[0:ofs0, :])
    yflat_ref[ofs0:ofs0 + Rp, :] = (
        jnp.where(valid, y1, 0.0).astype(yflat_ref.dtype))
    yflat_ref[ofs0 + Rp:, :] = jnp.zeros_like(yflat_ref[ofs0 + Rp:, :])

    z2 = None
    for dy in range(3):
        for dx in range(3):
            t = yflat_ref[dy * Wp + dx:dy * Wp + dx + Rp, :]
            d = jnp.dot(t, w2_ref[dy * 3 + dx].astype(jnp.bfloat16),
                        preferred_element_type=jnp.float32)
            z2 = d if z2 is None else z2 + d
    zv = jnp.where(valid, z2, 0.0)
    z2_ref[...] = zv.reshape(z2_ref.shape).astype(z2_ref.dtype)
    st_ref[0, 0:1, :] = jnp.sum(zv, axis=0, keepdims=True)
    st_ref[0, 1:2, :] = jnp.sum(zv * zv, axis=0, keepdims=True)
    st_ref[0, 2:8, :] = jnp.zeros_like(st_ref[0, 2:8, :])


def _final_kernel(z2_ref, zd_ref, s1_ref, s2_ref, g2_ref, b2_ref,
                  gd_ref, bd_ref, o_ref, *,
                  Nb, Ho, Wo, inv_m):
    Wp, Hp = Wo + 2, Ho + 2
    P = Hp * Wp
    Rp = Nb * P
    Lq = Rp - 2 * Wp

    s1 = jnp.sum(s1_ref[...], axis=0)
    s2 = jnp.sum(s2_ref[...], axis=0)

    z2f = z2_ref[...].reshape(Rp, -1)
    zdf = zd_ref[...].reshape(Rp, -1)

    m2 = s2[0:1, :] * inv_m
    v2 = jnp.maximum(s2[1:2, :] * inv_m - m2 * m2, 0.0)
    sc2 = jax.lax.rsqrt(v2 + EPS) * g2_ref[...]
    y2 = z2f[0:Lq, :].astype(jnp.float32) * sc2 + (b2_ref[...] - m2 * sc2)

    md = s1[0:1, :] * inv_m
    vd = jnp.maximum(s1[1:2, :] * inv_m - md * md, 0.0)
    scd = jax.lax.rsqrt(vd + EPS) * gd_ref[...]
    yd = (zdf[0:Lq, :].astype(jnp.float32) * scd
          + (bd_ref[...] - md * scd))

    o = jnp.maximum(y2 + yd, 0.0)
    for k in range(Nb):
        v = o[k * P:k * P + Ho * Wp, :].reshape(Ho, Wp, -1)[:, 0:Wo, :]
        o_ref[k * Ho * Wo:(k + 1) * Ho * Wo, :] = v.reshape(Ho * Wo, -1)


def _block_impl(x, w1, g1, b1, w2, g2, b2, wd, gd, bd):
    N, Cin, H, W = x.shape
    C = w1.shape[-1]
    assert H % 2 == 0 and W % 2 == 0
    assert Cin % 128 == 0 and C % 128 == 0
    Ho, Wo = H // 2, W // 2
    Ap = Ho + 1
    M = N * Ho * Wo
    inv_m = 1.0 / M
    Nb = next(t for t in (8, 4, 2, 1) if N % t == 0)
    nb = N // Nb
    R = Nb * Ho * Wo
    Hp, Wp = Ho + 2, Wo + 2
    P = Hp * Wp
    Rp = Nb * P

    xt = jnp.transpose(x.astype(jnp.bfloat16), (0, 2, 3, 1))

    w1r = w1.reshape(9, Cin, C)
    wdr = wd.reshape(Cin, C)
    w2r = w2.reshape(9, C, C)
    g1r, b1r = g1.reshape(1, C), b1.reshape(1, C)
    g2r, b2r = g2.reshape(1, C), b2.reshape(1, C)
    gdr, bdr = gd.reshape(1, C), bd.reshape(1, C)

    par = pltpu.CompilerParams(dimension_semantics=("parallel",),
                               vmem_limit_bytes=_VMEM_LIMIT)

    z, s1b = pl.pallas_call(
        functools.partial(_conv1_kernel, Nb=Nb, Ho=Ho, Wo=Wo, Cin=Cin),
        grid=(nb,),
        in_specs=[
            pl.BlockSpec((Nb, H, W, Cin), lambda i: (i, 0, 0, 0)),
            pl.BlockSpec((9, Cin, C), lambda i: (0, 0, 0)),
            pl.BlockSpec((Cin, C), lambda i: (0, 0)),
        ],
        out_specs=[
            pl.BlockSpec((Nb, Hp, Wp, 2 * C), lambda i: (i, 0, 0, 0)),
            pl.BlockSpec((1, 8, 2 * C), lambda i: (i, 0, 0)),
        ],
        out_shape=[
            jax.ShapeDtypeStruct((N, Hp, Wp, 2 * C), jnp.bfloat16),
            jax.ShapeDtypeStruct((nb, 8, 2 * C), jnp.float32),
        ],
        scratch_shapes=[
            pltpu.VMEM((Nb, 2 * Ap, 2 * Ap, Cin), jnp.bfloat16),
            pltpu.VMEM((4, Nb, Hp, Wp, Cin), jnp.bfloat16),
        ],
        compiler_params=par,
    )(xt, w1r, wdr)

    z2, s2b = pl.pallas_call(
        functools.partial(_conv2_kernel, Nb=Nb, Ho=Ho, Wo=Wo, C=C, inv_m=inv_m),
        grid=(nb,),
        in_specs=[
            pl.BlockSpec((Nb, Hp, Wp, C), lambda i: (i, 0, 0, 0)),
            pl.BlockSpec((9, C, C), lambda i: (0, 0, 0)),
            pl.BlockSpec((1, C), lambda i: (0, 0)),
            pl.BlockSpec((1, C), lambda i: (0, 0)),
            pl.BlockSpec((nb, 8, C), lambda i: (0, 0, 0)),
        ],
        out_specs=[
            pl.BlockSpec((Nb, Hp, Wp, C), lambda i: (i, 0, 0, 0)),
            pl.BlockSpec((1, 8, C), lambda i: (i, 0, 0)),
        ],
        out_shape=[
            jax.ShapeDtypeStruct((N, Hp, Wp, C), jnp.bfloat16),
            jax.ShapeDtypeStruct((nb, 8, C), jnp.float32),
        ],
        scratch_shapes=[
            pltpu.VMEM((Rp + 3 * Wp + 8, C), jnp.bfloat16)],
        compiler_params=par,
    )(z, w2r, g1r, b1r, s1b)

    out_flat = pl.pallas_call(
        functools.partial(_final_kernel, Nb=Nb, Ho=Ho, Wo=Wo, inv_m=inv_m),
        grid=(nb,),
        in_specs=[
            pl.BlockSpec((Nb, Hp, Wp, C), lambda i: (i, 0, 0, 0)),
            pl.BlockSpec((Nb, Hp, Wp, C), lambda i: (i, 0, 0, 1)),
            pl.BlockSpec((nb, 8, C), lambda i: (0, 0, 1)),
            pl.BlockSpec((nb, 8, C), lambda i: (0, 0, 0)),
            pl.BlockSpec((1, C), lambda i: (0, 0)),
            pl.BlockSpec((1, C), lambda i: (0, 0)),
            pl.BlockSpec((1, C), lambda i: (0, 0)),
            pl.BlockSpec((1, C), lambda i: (0, 0)),
        ],
        out_specs=pl.BlockSpec((R, C), lambda i: (i, 0)),
        out_shape=jax.ShapeDtypeStruct((M, C), jnp.float32),
        compiler_params=par,
    )(z2, z, s1b, s2b, g2r, b2r, gdr, bdr)

    return out_flat.reshape(N, Ho, Wo, C).transpose(0, 3, 1, 2)


def kernel(x, w1, g1, b1, w2, g2, b2, wd, gd, bd):
    return _block_impl(x, w1, g1, b1, w2, g2, b2, wd, gd, bd)
